# baseline (speedup 1.0000x reference)
import jax
import jax.numpy as jnp
from jax import lax
from jax.experimental import pallas as pl
from jax.experimental.pallas import tpu as pltpu
from jax.experimental.pallas import tpu_sc as plsc

HIDDIM = 128
N_NODES = 10000
N_EDGES = 320000
N_TUPLES = 320000

NC = 2
NS = 16
NW = NC * NS

CHUNK = 400


def _gather_chunk(idx_v, table_v, out_hbm, rows_v, sem, idx_off, start, n):
    pltpu.async_copy(table_v.at[idx_v.at[pl.ds(idx_off, n)]],
                     rows_v.at[pl.ds(0, n)], sem).wait()
    pltpu.sync_copy(rows_v.at[pl.ds(0, n)], out_hbm.at[pl.ds(start, n)])


def _gather_stream(idx_hbm, table_v, out_hbm, idx_v, rows_v, sem,
                   base, count):
    pltpu.sync_copy(idx_hbm.at[pl.ds(base, count)],
                    idx_v.at[pl.ds(0, count)])
    n_chunks = count // CHUNK
    rem = count % CHUNK

    def body(j, carry):
        _gather_chunk(idx_v, table_v, out_hbm, rows_v, sem,
                      j * CHUNK, base + j * CHUNK, CHUNK)
        return carry

    if n_chunks:
        lax.fori_loop(0, n_chunks, body, 0, unroll=False)
    if rem:
        _gather_chunk(idx_v, table_v, out_hbm, rows_v, sem,
                      n_chunks * CHUNK, base + n_chunks * CHUNK, rem)


def _sc_body(x_hbm, a_hbm, t_hbm, x_table_hbm, ea_table_hbm,
             tuple_table_hbm, x_out, a_out, t_out,
             idx_v, rows_v, xtab_v, etab_v, ttab_v, sem):
    wid = lax.axis_index("s") * NC + lax.axis_index("c")

    @pl.when(lax.axis_index("s") == 0)
    def _():
        pltpu.sync_copy(x_table_hbm, xtab_v)
        pltpu.sync_copy(ea_table_hbm, etab_v)
        pltpu.sync_copy(tuple_table_hbm, ttab_v)

    plsc.subcore_barrier()

    x_per_w = N_NODES // NW // 8 * 8
    _gather_stream(x_hbm, xtab_v, x_out, idx_v, rows_v, sem,
                   wid * x_per_w, x_per_w)
    x_rem = N_NODES - NW * x_per_w

    @pl.when(wid == NW - 1)
    def _():
        pltpu.sync_copy(x_hbm.at[pl.ds(NW * x_per_w, x_rem)],
                        idx_v.at[pl.ds(0, x_rem)])
        _gather_chunk(idx_v, xtab_v, x_out, rows_v, sem,
                      0, NW * x_per_w, x_rem)

    e_per_w = N_EDGES // NW
    _gather_stream(a_hbm, etab_v, a_out, idx_v, rows_v, sem,
                   wid * e_per_w, e_per_w)
    _gather_stream(t_hbm, ttab_v, t_out, idx_v, rows_v, sem,
                   wid * e_per_w, e_per_w)


@jax.jit
def _encode(x, A_values, X_values, x_table, ea_table, tuple_table):
    mesh = plsc.VectorSubcoreMesh(core_axis_name="c", subcore_axis_name="s")
    run = pl.kernel(
        _sc_body,
        out_type=(
            jax.ShapeDtypeStruct((N_NODES, HIDDIM), jnp.float32),
            jax.ShapeDtypeStruct((N_EDGES, HIDDIM), jnp.float32),
            jax.ShapeDtypeStruct((N_TUPLES, HIDDIM), jnp.float32),
        ),
        mesh=mesh,
        scratch_types=[
            pltpu.VMEM((N_EDGES // NW,), jnp.int32),
            pltpu.VMEM((CHUNK, HIDDIM), jnp.float32),
            pltpu.MemorySpace.VMEM_SHARED((32, HIDDIM), jnp.float32),
            pltpu.MemorySpace.VMEM_SHARED((16, HIDDIM), jnp.float32),
            pltpu.MemorySpace.VMEM_SHARED((16, HIDDIM), jnp.float32),
            pltpu.SemaphoreType.DMA,
        ],
    )
    return run(x, A_values, X_values, x_table, ea_table, tuple_table)


def kernel(x, A_values, X_values, x_table, ea_table, tuple_table):
    return _encode(x.astype(jnp.int32).reshape(-1), A_values, X_values,
                   x_table, ea_table, tuple_table)


# speedup vs baseline: 1.4209x; 1.4209x over previous
import jax
import jax.numpy as jnp
from jax import lax
from jax.experimental import pallas as pl
from jax.experimental.pallas import tpu as pltpu
from jax.experimental.pallas import tpu_sc as plsc

HIDDIM = 128
N_NODES = 10000
N_EDGES = 320000
N_TUPLES = 320000

NC = 2
NS = 16
NW = NC * NS

CHUNK = 400


def _pipelined_stream(idx_hbm, table_s, out_hbm, idx_v, rows, gsems,
                      ssems, base, count):
    n = count // CHUNK
    assert n % 2 == 1 and count % CHUNK == 0

    pltpu.sync_copy(idx_hbm.at[pl.ds(base, count)],
                    idx_v.at[pl.ds(0, count)])

    def gather(j, b):
        pltpu.async_copy(table_s.at[idx_v.at[pl.ds(j * CHUNK, CHUNK)]],
                         rows[b], gsems[b])

    def gather_wait(b):
        pltpu.make_async_copy(out_hbm.at[pl.ds(0, CHUNK)], rows[b],
                              gsems[b]).wait()

    def scatter(j, b):
        pltpu.async_copy(rows[b],
                         out_hbm.at[pl.ds(base + j * CHUNK, CHUNK)],
                         ssems[b])

    def scatter_wait(b):
        pltpu.make_async_copy(rows[b], out_hbm.at[pl.ds(0, CHUNK)],
                              ssems[b]).wait()

    gather(0, 0)
    gather_wait(0)
    scatter(0, 0)
    gather(1, 1)

    def body(g, carry):
        j = 1 + 2 * g
        gather_wait(1)
        scatter(j, 1)
        scatter_wait(0)
        gather(j + 1, 0)
        gather_wait(0)
        scatter(j + 1, 0)
        scatter_wait(1)

        @pl.when(g + 1 < (n - 1) // 2)
        def _():
            gather(j + 2, 1)

        return carry

    lax.fori_loop(0, (n - 1) // 2, body, 0, unroll=False)
    scatter_wait(0)


def _simple_gather(idx_hbm, table_s, out_hbm, idx_v, rows, gsems, ssems,
                   start, m):
    pltpu.sync_copy(idx_hbm.at[pl.ds(start, m)], idx_v.at[pl.ds(0, m)])
    pltpu.async_copy(table_s.at[idx_v.at[pl.ds(0, m)]],
                     rows[0].at[pl.ds(0, m)], gsems[0]).wait()
    pltpu.sync_copy(rows[0].at[pl.ds(0, m)], out_hbm.at[pl.ds(start, m)])


def _sc_body(x_hbm, a_hbm, t_hbm, x_table_hbm, ea_table_hbm,
             tuple_table_hbm, x_out, a_out, t_out,
             idx_v, rows0, rows1, xtab_s, etab_s, ttab_s,
             gsem0, gsem1, ssem0, ssem1):
    wid = lax.axis_index("s") * NC + lax.axis_index("c")
    rows = (rows0, rows1)
    gsems = (gsem0, gsem1)
    ssems = (ssem0, ssem1)

    @pl.when(lax.axis_index("s") == 0)
    def _():
        pltpu.sync_copy(x_table_hbm, xtab_s)
        pltpu.sync_copy(ea_table_hbm, etab_s)
        pltpu.sync_copy(tuple_table_hbm, ttab_s)

    plsc.subcore_barrier()

    x_per_w = N_NODES // NW // 8 * 8
    _simple_gather(x_hbm, xtab_s, x_out, idx_v, rows, gsems, ssems,
                   wid * x_per_w, x_per_w)
    x_rem = N_NODES - NW * x_per_w

    @pl.when(wid == NW - 1)
    def _():
        _simple_gather(x_hbm, xtab_s, x_out, idx_v, rows, gsems, ssems,
                       NW * x_per_w, x_rem)

    e_per_w = N_EDGES // NW
    _pipelined_stream(a_hbm, etab_s, a_out, idx_v, rows, gsems, ssems,
                      wid * e_per_w, e_per_w)
    _pipelined_stream(t_hbm, ttab_s, t_out, idx_v, rows, gsems, ssems,
                      wid * e_per_w, e_per_w)


@jax.jit
def _encode(x, A_values, X_values, x_table, ea_table, tuple_table):
    mesh = plsc.VectorSubcoreMesh(core_axis_name="c", subcore_axis_name="s")
    run = pl.kernel(
        _sc_body,
        out_type=(
            jax.ShapeDtypeStruct((N_NODES, HIDDIM), jnp.float32),
            jax.ShapeDtypeStruct((N_EDGES, HIDDIM), jnp.float32),
            jax.ShapeDtypeStruct((N_TUPLES, HIDDIM), jnp.float32),
        ),
        mesh=mesh,
        scratch_types=[
            pltpu.VMEM((N_EDGES // NW,), jnp.int32),
            pltpu.VMEM((CHUNK, HIDDIM), jnp.float32),
            pltpu.VMEM((CHUNK, HIDDIM), jnp.float32),
            pltpu.MemorySpace.VMEM_SHARED((32, HIDDIM), jnp.float32),
            pltpu.MemorySpace.VMEM_SHARED((16, HIDDIM), jnp.float32),
            pltpu.MemorySpace.VMEM_SHARED((16, HIDDIM), jnp.float32),
            pltpu.SemaphoreType.DMA,
            pltpu.SemaphoreType.DMA,
            pltpu.SemaphoreType.DMA,
            pltpu.SemaphoreType.DMA,
        ],
    )
    return run(x, A_values, X_values, x_table, ea_table, tuple_table)


def kernel(x, A_values, X_values, x_table, ea_table, tuple_table):
    return _encode(x.astype(jnp.int32).reshape(-1), A_values, X_values,
                   x_table, ea_table, tuple_table)


# speedup vs baseline: 1.4482x; 1.0193x over previous
import jax
import jax.numpy as jnp
from jax import lax
from jax.experimental import pallas as pl
from jax.experimental.pallas import tpu as pltpu
from jax.experimental.pallas import tpu_sc as plsc

HIDDIM = 128
N_NODES = 10000
N_EDGES = 320000
N_TUPLES = 320000

NC = 2
NS = 16
NW = NC * NS

CHUNK = 200
NBUF = 4


def _pipelined_stream(table_s, out_hbm, idx_v, rows, gsems, ssems, base,
                      count):
    n = count // CHUNK
    assert n % CHUNK == 0 or (n - 2) % NBUF == 0

    def gather(j, b):
        pltpu.async_copy(table_s.at[idx_v.at[pl.ds(j * CHUNK, CHUNK)]],
                         rows[b], gsems[b])

    def gather_wait(b):
        pltpu.make_async_copy(out_hbm.at[pl.ds(0, CHUNK)], rows[b],
                              gsems[b]).wait()

    def scatter(j, b):
        pltpu.async_copy(rows[b],
                         out_hbm.at[pl.ds(base + j * CHUNK, CHUNK)],
                         ssems[b])

    def scatter_wait(b):
        pltpu.make_async_copy(rows[b], out_hbm.at[pl.ds(0, CHUNK)],
                              ssems[b]).wait()

    gather(0, 0)
    gather(1, 1)
    gather_wait(0)
    scatter(0, 0)
    gather(2, 2)
    gather_wait(1)
    scatter(1, 1)
    gather(3, 3)

    def body(g, carry):
        for k in range(NBUF):
            j = 2 + g * NBUF + k
            b = (2 + k) % NBUF
            b2 = k % NBUF
            gather_wait(b)
            scatter(j, b)
            scatter_wait(b2)

            @pl.when(j + 2 < n)
            def _():
                gather(j + 2, b2)

        return carry

    lax.fori_loop(0, (n - 2) // NBUF, body, 0, unroll=False)
    scatter_wait((n - 2) % NBUF)
    scatter_wait((n - 1) % NBUF)


def _simple_gather(idx_v, table_s, out_hbm, rows, gsems, idx_off, start,
                   m):
    pltpu.async_copy(table_s.at[idx_v.at[pl.ds(idx_off, m)]],
                     rows[0].at[pl.ds(0, m)], gsems[0])
    pltpu.make_async_copy(out_hbm.at[pl.ds(0, m)],
                          rows[0].at[pl.ds(0, m)], gsems[0]).wait()
    pltpu.sync_copy(rows[0].at[pl.ds(0, m)], out_hbm.at[pl.ds(start, m)])


def _sc_body(x_hbm, a_hbm, t_hbm, x_table_hbm, ea_table_hbm,
             tuple_table_hbm, x_out, a_out, t_out,
             idx_x, idx_a, idx_t, rows0, rows1, rows2, rows3,
             xtab_s, etab_s, ttab_s,
             gsem0, gsem1, gsem2, gsem3, ssem0, ssem1, ssem2, ssem3,
             isem_a, isem_t):
    wid = lax.axis_index("s") * NC + lax.axis_index("c")
    rows = (rows0, rows1, rows2, rows3)
    gsems = (gsem0, gsem1, gsem2, gsem3)
    ssems = (ssem0, ssem1, ssem2, ssem3)
    e_per_w = N_EDGES // NW

    a_idx_copy = pltpu.make_async_copy(
        a_hbm.at[pl.ds(wid * e_per_w, e_per_w)], idx_a, isem_a)
    a_idx_copy.start()
    t_idx_copy = pltpu.make_async_copy(
        t_hbm.at[pl.ds(wid * e_per_w, e_per_w)], idx_t, isem_t)
    t_idx_copy.start()

    @pl.when(lax.axis_index("s") == 0)
    def _():
        pltpu.sync_copy(x_table_hbm, xtab_s)
        pltpu.sync_copy(ea_table_hbm, etab_s)
        pltpu.sync_copy(tuple_table_hbm, ttab_s)

    plsc.subcore_barrier()

    x_per_w = N_NODES // NW // 8 * 8
    pltpu.sync_copy(x_hbm.at[pl.ds(wid * x_per_w, x_per_w)],
                    idx_x.at[pl.ds(0, x_per_w)])
    _simple_gather(idx_x, xtab_s, x_out, rows, gsems, 0,
                   wid * x_per_w, 160)
    _simple_gather(idx_x, xtab_s, x_out, rows, gsems, 160,
                   wid * x_per_w + 160, x_per_w - 160)
    x_rem = N_NODES - NW * x_per_w

    @pl.when(wid == NW - 1)
    def _():
        pltpu.sync_copy(x_hbm.at[pl.ds(NW * x_per_w, x_rem)],
                        idx_x.at[pl.ds(0, x_rem)])
        _simple_gather(idx_x, xtab_s, x_out, rows, gsems, 0,
                       NW * x_per_w, x_rem)

    a_idx_copy.wait()
    _pipelined_stream(etab_s, a_out, idx_a, rows, gsems, ssems,
                      wid * e_per_w, e_per_w)
    t_idx_copy.wait()
    _pipelined_stream(ttab_s, t_out, idx_t, rows, gsems, ssems,
                      wid * e_per_w, e_per_w)


@jax.jit
def _encode(x, A_values, X_values, x_table, ea_table, tuple_table):
    mesh = plsc.VectorSubcoreMesh(core_axis_name="c", subcore_axis_name="s")
    run = pl.kernel(
        _sc_body,
        out_type=(
            jax.ShapeDtypeStruct((N_NODES, HIDDIM), jnp.float32),
            jax.ShapeDtypeStruct((N_EDGES, HIDDIM), jnp.float32),
            jax.ShapeDtypeStruct((N_TUPLES, HIDDIM), jnp.float32),
        ),
        mesh=mesh,
        scratch_types=[
            pltpu.VMEM((N_NODES // NW // 8 * 8 + 16,), jnp.int32),
            pltpu.VMEM((N_EDGES // NW,), jnp.int32),
            pltpu.VMEM((N_TUPLES // NW,), jnp.int32),
            pltpu.VMEM((CHUNK, HIDDIM), jnp.float32),
            pltpu.VMEM((CHUNK, HIDDIM), jnp.float32),
            pltpu.VMEM((CHUNK, HIDDIM), jnp.float32),
            pltpu.VMEM((CHUNK, HIDDIM), jnp.float32),
            pltpu.MemorySpace.VMEM_SHARED((32, HIDDIM), jnp.float32),
            pltpu.MemorySpace.VMEM_SHARED((16, HIDDIM), jnp.float32),
            pltpu.MemorySpace.VMEM_SHARED((16, HIDDIM), jnp.float32),
            pltpu.SemaphoreType.DMA,
            pltpu.SemaphoreType.DMA,
            pltpu.SemaphoreType.DMA,
            pltpu.SemaphoreType.DMA,
            pltpu.SemaphoreType.DMA,
            pltpu.SemaphoreType.DMA,
            pltpu.SemaphoreType.DMA,
            pltpu.SemaphoreType.DMA,
            pltpu.SemaphoreType.DMA,
            pltpu.SemaphoreType.DMA,
        ],
    )
    return run(x, A_values, X_values, x_table, ea_table, tuple_table)


def kernel(x, A_values, X_values, x_table, ea_table, tuple_table):
    return _encode(x.astype(jnp.int32).reshape(-1), A_values, X_values,
                   x_table, ea_table, tuple_table)
